# Initial kernel scaffold; baseline (speedup 1.0000x reference)
import jax
import jax.numpy as jnp
from jax.experimental import pallas as pl


def kernel(x, edge_index, batch, W1, b1, g1, be1, W2, b2, g2, be2, Wl, bl):
    raise NotImplementedError("write your pallas kernel here")


# speedup vs baseline: 16.6744x; 16.6744x over previous
import functools

import jax
import jax.numpy as jnp
from jax import lax
from jax.experimental import pallas as pl
from jax.experimental.pallas import tpu as pltpu
from jax.experimental.pallas import tpu_sc as plsc

N = 10000
E = 320000
D = 128
HD = D // 2
G = 64
C = 40
EPS = 1e-5

NC = 2
NS = 16
NW = NC * NS
L = 16

CHUNK = 128
CPT = 80
EPAD = NW * CPT * CHUNK
ACCN = 10112
PADROWS = ACCN - N
RPT = ACCN // NS
OUT_TAIL = N - RPT * (NS - 1)


def _mesh():
    return plsc.VectorSubcoreMesh(
        core_axis_name="c", subcore_axis_name="s", num_cores=NC, num_subcores=NS)


def _pad_edges(idx, sink):
    pad = (sink + jnp.arange(EPAD - E, dtype=idx.dtype) % PADROWS
           if sink is not None else
           (jnp.arange(EPAD - E, dtype=idx.dtype) * 131) % N)
    return jnp.concatenate([idx, pad]).reshape(EPAD // CHUNK, CHUNK)


def _sc_degree(dst2d):
    @functools.partial(
        pl.kernel,
        out_type=jax.ShapeDtypeStruct((NC, N), jnp.float32),
        mesh=_mesh(),
        compiler_params=pltpu.CompilerParams(use_tc_tiling_on_sc=False),
        scratch_types=[
            pltpu.VMEM((CPT, CHUNK), jnp.int32),
            pltpu.VMEM((CHUNK,), jnp.float32),
            pltpu.VMEM((RPT,), jnp.float32),
            pltpu.VMEM_SHARED((ACCN,), jnp.float32),
        ],
    )
    def k(dst_hbm, out_hbm, dsts, ones, zbuf, acc):
        c = lax.axis_index("c")
        s = lax.axis_index("s")
        wid = c * NS + s

        one16 = jnp.ones((L,), jnp.float32)
        zero16 = jnp.zeros((L,), jnp.float32)
        for kk in range(CHUNK // L):
            ones[pl.ds(kk * L, L)] = one16
        for kk in range(RPT // L):
            zbuf[pl.ds(kk * L, L)] = zero16

        pltpu.sync_copy(zbuf, acc.at[pl.ds(s * RPT, RPT)])
        plsc.subcore_barrier()

        pltpu.sync_copy(dst_hbm.at[pl.ds(wid * CPT, CPT)], dsts)

        def body(j, carry):
            pltpu.sync_copy(ones, acc.at[dsts.at[j]], add=True)
            return carry

        lax.fori_loop(0, CPT, body, 0)
        plsc.subcore_barrier()

        @pl.when(s < NS - 1)
        def _():
            pltpu.sync_copy(acc.at[pl.ds(s * RPT, RPT)],
                            out_hbm.at[c, pl.ds(s * RPT, RPT)])

        @pl.when(s == NS - 1)
        def _():
            pltpu.sync_copy(acc.at[pl.ds((NS - 1) * RPT, OUT_TAIL)],
                            out_hbm.at[c, pl.ds((NS - 1) * RPT, OUT_TAIL)])

    return k(dst2d)


def _sc_propagate(uA, uB, src2d, dst2d):
    @functools.partial(
        pl.kernel,
        out_type=jax.ShapeDtypeStruct((2, NC, N, HD), jnp.float32),
        mesh=_mesh(),
        compiler_params=pltpu.CompilerParams(use_tc_tiling_on_sc=False),
        scratch_types=[
            pltpu.VMEM((CPT, CHUNK), jnp.int32),
            pltpu.VMEM((CPT, CHUNK), jnp.int32),
            pltpu.VMEM((CHUNK, HD), jnp.float32),
            pltpu.VMEM((RPT, HD), jnp.float32),
            pltpu.VMEM_SHARED((ACCN, HD), jnp.float32),
        ],
    )
    def k(uA_hbm, uB_hbm, src_hbm, dst_hbm, out_hbm, srcs, dsts, rows, zbuf,
          acc):
        c = lax.axis_index("c")
        s = lax.axis_index("s")
        wid = c * NS + s

        zero16 = jnp.zeros((L,), jnp.float32)

        def zrow(r, carry):
            for kk in range(HD // L):
                zbuf[r, pl.ds(kk * L, L)] = zero16
            return carry

        lax.fori_loop(0, RPT, zrow, 0)

        pltpu.sync_copy(src_hbm.at[pl.ds(wid * CPT, CPT)], srcs)
        pltpu.sync_copy(dst_hbm.at[pl.ds(wid * CPT, CPT)], dsts)

        for half, u_hbm in ((0, uA_hbm), (1, uB_hbm)):
            pltpu.sync_copy(zbuf, acc.at[pl.ds(s * RPT, RPT)])
            plsc.subcore_barrier()

            def body(j, carry):
                pltpu.sync_copy(u_hbm.at[srcs.at[j]], rows)
                pltpu.sync_copy(rows, acc.at[dsts.at[j]], add=True)
                return carry

            lax.fori_loop(0, CPT, body, 0)
            plsc.subcore_barrier()

            @pl.when(s < NS - 1)
            def _():
                pltpu.sync_copy(acc.at[pl.ds(s * RPT, RPT)],
                                out_hbm.at[half, c, pl.ds(s * RPT, RPT)])

            @pl.when(s == NS - 1)
            def _():
                pltpu.sync_copy(
                    acc.at[pl.ds((NS - 1) * RPT, OUT_TAIL)],
                    out_hbm.at[half, c, pl.ds((NS - 1) * RPT, OUT_TAIL)])

            plsc.subcore_barrier()

    return k(uA, uB, src2d, dst2d)


def _split(u):
    return u[:, :HD], u[:, HD:]


def _combine(p_ref, h, dv, b):
    sA = p_ref[0, 0] + p_ref[0, 1]
    sB = p_ref[1, 0] + p_ref[1, 1]
    return dv * jnp.concatenate([sA, sB], axis=1) + (dv * dv) * h + b


def _tc_first(x, W1, degT):
    def body(x_ref, w_ref, deg_ref, h_ref, uA_ref, uB_ref, dinv_ref):
        deg = jnp.sum(deg_ref[...], axis=1, keepdims=True) + 1.0
        dinv = lax.rsqrt(deg)
        h = jnp.dot(x_ref[...], w_ref[...], preferred_element_type=jnp.float32)
        u = h * dinv
        h_ref[...] = h
        uA_ref[...], uB_ref[...] = _split(u)
        dinv_ref[...] = dinv

    return pl.pallas_call(
        body,
        out_shape=(
            jax.ShapeDtypeStruct((N, D), jnp.float32),
            jax.ShapeDtypeStruct((N, HD), jnp.float32),
            jax.ShapeDtypeStruct((N, HD), jnp.float32),
            jax.ShapeDtypeStruct((N, 1), jnp.float32),
        ),
    )(x, W1, degT)


def _bn_relu(a, g, be):
    mu = jnp.mean(a, axis=0, keepdims=True)
    d = a - mu
    var = jnp.mean(d * d, axis=0, keepdims=True)
    return jnp.maximum(d * lax.rsqrt(var + EPS) * g + be, 0.0)


def _tc_mid(p, h1, dinv, b1, g1, be1, W2):
    def body(p_ref, h_ref, dinv_ref, b_ref, g_ref, be_ref, w_ref,
             h2_ref, u2A_ref, u2B_ref):
        dv = dinv_ref[...]
        a = _combine(p_ref, h_ref[...], dv, b_ref[...])
        y = _bn_relu(a, g_ref[...], be_ref[...])
        h2 = jnp.dot(y, w_ref[...], preferred_element_type=jnp.float32)
        u2 = h2 * dv
        h2_ref[...] = h2
        u2A_ref[...], u2B_ref[...] = _split(u2)

    return pl.pallas_call(
        body,
        out_shape=(
            jax.ShapeDtypeStruct((N, D), jnp.float32),
            jax.ShapeDtypeStruct((N, HD), jnp.float32),
            jax.ShapeDtypeStruct((N, HD), jnp.float32),
        ),
    )(p, h1, dinv, b1, g1, be1, W2)


def _tc_final(p, h2, dinv, b2, g2, be2, batch2d, Wl, bl):
    def body(p_ref, h_ref, dinv_ref, b_ref, g_ref, be_ref, bat_ref,
             wl_ref, bl_ref, o_ref):
        dv = dinv_ref[...]
        a = _combine(p_ref, h_ref[...], dv, b_ref[...])
        y = _bn_relu(a, g_ref[...], be_ref[...])
        oh = (bat_ref[...] == lax.broadcasted_iota(jnp.int32, (N, G), 1)
              ).astype(jnp.float32)
        dnums = (((0,), (0,)), ((), ()))
        sums = lax.dot_general(oh, y, dnums,
                               preferred_element_type=jnp.float32)
        cnt = lax.dot_general(oh, jnp.ones((N, 1), jnp.float32), dnums,
                              preferred_element_type=jnp.float32)
        pooled = sums / jnp.maximum(cnt, 1.0)
        o_ref[...] = jnp.dot(pooled, wl_ref[...],
                             preferred_element_type=jnp.float32) + bl_ref[...]

    return pl.pallas_call(
        body,
        out_shape=jax.ShapeDtypeStruct((G, C), jnp.float32),
    )(p, h2, dinv, b2, g2, be2, batch2d, Wl, bl)


@jax.jit
def kernel(x, edge_index, batch, W1, b1, g1, be1, W2, b2, g2, be2, Wl, bl):
    src2d = _pad_edges(edge_index[0], None)
    dst2d = _pad_edges(edge_index[1], N)

    degp = _sc_degree(dst2d)
    degT = degp.T
    h1, u1A, u1B, dinv = _tc_first(x, W1, degT)
    p1 = _sc_propagate(u1A, u1B, src2d, dst2d)
    h2, u2A, u2B = _tc_mid(p1, h1, dinv,
                           b1.reshape(1, D), g1.reshape(1, D),
                           be1.reshape(1, D), W2)
    p2 = _sc_propagate(u2A, u2B, src2d, dst2d)
    return _tc_final(p2, h2, dinv,
                     b2.reshape(1, D), g2.reshape(1, D), be2.reshape(1, D),
                     batch.reshape(N, 1), Wl, bl.reshape(1, C))


# speedup vs baseline: 20.6131x; 1.2362x over previous
import functools

import jax
import jax.numpy as jnp
from jax import lax
from jax.experimental import pallas as pl
from jax.experimental.pallas import tpu as pltpu
from jax.experimental.pallas import tpu_sc as plsc

N = 10000
E = 320000
D = 128
HD = D // 2
G = 64
C = 40
EPS = 1e-5

NC = 2
NS = 16
NW = NC * NS
L = 16

CHUNK = 128
CPT = 80
EPAD = NW * CPT * CHUNK
ACCN = 10112
PADROWS = ACCN - N
RPT = ACCN // NS
OUT_TAIL = N - RPT * (NS - 1)
NBUF = 2


def _mesh():
    return plsc.VectorSubcoreMesh(
        core_axis_name="c", subcore_axis_name="s", num_cores=NC, num_subcores=NS)


def _pad_edges(idx, sink):
    pad = (sink + jnp.arange(EPAD - E, dtype=idx.dtype) % PADROWS
           if sink is not None else
           (jnp.arange(EPAD - E, dtype=idx.dtype) * 131) % N)
    return jnp.concatenate([idx, pad]).reshape(EPAD // CHUNK, CHUNK)


def _sc_degree(dst2d):
    @functools.partial(
        pl.kernel,
        out_type=jax.ShapeDtypeStruct((NC, N), jnp.float32),
        mesh=_mesh(),
        compiler_params=pltpu.CompilerParams(use_tc_tiling_on_sc=False),
        scratch_types=[
            pltpu.VMEM((CPT, CHUNK), jnp.int32),
            pltpu.VMEM((CHUNK,), jnp.float32),
            pltpu.VMEM((RPT,), jnp.float32),
            pltpu.VMEM_SHARED((ACCN,), jnp.float32),
        ],
    )
    def k(dst_hbm, out_hbm, dsts, ones, zbuf, acc):
        c = lax.axis_index("c")
        s = lax.axis_index("s")
        wid = c * NS + s

        one16 = jnp.ones((L,), jnp.float32)
        zero16 = jnp.zeros((L,), jnp.float32)
        for kk in range(CHUNK // L):
            ones[pl.ds(kk * L, L)] = one16
        for kk in range(RPT // L):
            zbuf[pl.ds(kk * L, L)] = zero16

        pltpu.sync_copy(zbuf, acc.at[pl.ds(s * RPT, RPT)])
        plsc.subcore_barrier()

        pltpu.sync_copy(dst_hbm.at[pl.ds(wid * CPT, CPT)], dsts)

        def body(j, carry):
            pltpu.sync_copy(ones, acc.at[dsts.at[j]], add=True)
            return carry

        lax.fori_loop(0, CPT, body, 0)
        plsc.subcore_barrier()

        @pl.when(s < NS - 1)
        def _():
            pltpu.sync_copy(acc.at[pl.ds(s * RPT, RPT)],
                            out_hbm.at[c, pl.ds(s * RPT, RPT)])

        @pl.when(s == NS - 1)
        def _():
            pltpu.sync_copy(acc.at[pl.ds((NS - 1) * RPT, OUT_TAIL)],
                            out_hbm.at[c, pl.ds((NS - 1) * RPT, OUT_TAIL)])

    return k(dst2d)


def _sc_propagate(uA, uB, src2d, dst2d):
    @functools.partial(
        pl.kernel,
        out_type=jax.ShapeDtypeStruct((2, NC, N, HD), jnp.float32),
        mesh=_mesh(),
        compiler_params=pltpu.CompilerParams(use_tc_tiling_on_sc=False),
        scratch_types=[
            pltpu.VMEM((CPT, CHUNK), jnp.int32),
            pltpu.VMEM((CPT, CHUNK), jnp.int32),
            pltpu.VMEM((NBUF, CHUNK, HD), jnp.float32),
            pltpu.VMEM((RPT, HD), jnp.float32),
            pltpu.VMEM_SHARED((ACCN, HD), jnp.float32),
            pltpu.SemaphoreType.DMA,
        ] + [pltpu.SemaphoreType.DMA] * NBUF,
    )
    def k(uA_hbm, uB_hbm, src_hbm, dst_hbm, out_hbm, srcs, dsts, rows, zbuf,
          acc, ssem, *gsems):
        c = lax.axis_index("c")
        s = lax.axis_index("s")
        wid = c * NS + s

        zero16 = jnp.zeros((L,), jnp.float32)

        def zrow(r, carry):
            for kk in range(HD // L):
                zbuf[r, pl.ds(kk * L, L)] = zero16
            return carry

        lax.fori_loop(0, RPT, zrow, 0)

        pltpu.sync_copy(src_hbm.at[pl.ds(wid * CPT, CPT)], srcs)
        pltpu.sync_copy(dst_hbm.at[pl.ds(wid * CPT, CPT)], dsts)

        for half, u_hbm in ((0, uA_hbm), (1, uB_hbm)):
            pltpu.sync_copy(zbuf, acc.at[pl.ds(s * RPT, RPT)])
            plsc.subcore_barrier()

            def body(it, carry):
                j0 = it * NBUF
                for b in range(NBUF):
                    pltpu.async_copy(
                        u_hbm.at[srcs.at[j0 + b]], rows.at[b], gsems[b])
                for b in range(NBUF):
                    pltpu.make_async_copy(
                        u_hbm.at[srcs.at[j0 + b]], rows.at[b], gsems[b]).wait()
                for b in range(NBUF):
                    pltpu.async_copy(
                        rows.at[b], acc.at[dsts.at[j0 + b]], ssem, add=True)
                for b in range(NBUF):
                    pltpu.make_async_copy(
                        rows.at[b], acc.at[dsts.at[j0 + b]], ssem).wait()
                return carry

            lax.fori_loop(0, CPT // NBUF, body, 0)
            plsc.subcore_barrier()

            @pl.when(s < NS - 1)
            def _():
                pltpu.sync_copy(acc.at[pl.ds(s * RPT, RPT)],
                                out_hbm.at[half, c, pl.ds(s * RPT, RPT)])

            @pl.when(s == NS - 1)
            def _():
                pltpu.sync_copy(
                    acc.at[pl.ds((NS - 1) * RPT, OUT_TAIL)],
                    out_hbm.at[half, c, pl.ds((NS - 1) * RPT, OUT_TAIL)])

            plsc.subcore_barrier()

    return k(uA, uB, src2d, dst2d)


def _split(u):
    return u[:, :HD], u[:, HD:]


def _combine(p_ref, h, dv, b):
    sA = p_ref[0, 0] + p_ref[0, 1]
    sB = p_ref[1, 0] + p_ref[1, 1]
    return dv * jnp.concatenate([sA, sB], axis=1) + (dv * dv) * h + b


def _tc_first(x, W1, degT):
    def body(x_ref, w_ref, deg_ref, h_ref, uA_ref, uB_ref, dinv_ref):
        deg = jnp.sum(deg_ref[...], axis=1, keepdims=True) + 1.0
        dinv = lax.rsqrt(deg)
        h = jnp.dot(x_ref[...], w_ref[...], preferred_element_type=jnp.float32)
        u = h * dinv
        h_ref[...] = h
        uA_ref[...], uB_ref[...] = _split(u)
        dinv_ref[...] = dinv

    return pl.pallas_call(
        body,
        out_shape=(
            jax.ShapeDtypeStruct((N, D), jnp.float32),
            jax.ShapeDtypeStruct((N, HD), jnp.float32),
            jax.ShapeDtypeStruct((N, HD), jnp.float32),
            jax.ShapeDtypeStruct((N, 1), jnp.float32),
        ),
    )(x, W1, degT)


def _bn_relu(a, g, be):
    mu = jnp.mean(a, axis=0, keepdims=True)
    d = a - mu
    var = jnp.mean(d * d, axis=0, keepdims=True)
    return jnp.maximum(d * lax.rsqrt(var + EPS) * g + be, 0.0)


def _tc_mid(p, h1, dinv, b1, g1, be1, W2):
    def body(p_ref, h_ref, dinv_ref, b_ref, g_ref, be_ref, w_ref,
             h2_ref, u2A_ref, u2B_ref):
        dv = dinv_ref[...]
        a = _combine(p_ref, h_ref[...], dv, b_ref[...])
        y = _bn_relu(a, g_ref[...], be_ref[...])
        h2 = jnp.dot(y, w_ref[...], preferred_element_type=jnp.float32)
        u2 = h2 * dv
        h2_ref[...] = h2
        u2A_ref[...], u2B_ref[...] = _split(u2)

    return pl.pallas_call(
        body,
        out_shape=(
            jax.ShapeDtypeStruct((N, D), jnp.float32),
            jax.ShapeDtypeStruct((N, HD), jnp.float32),
            jax.ShapeDtypeStruct((N, HD), jnp.float32),
        ),
    )(p, h1, dinv, b1, g1, be1, W2)


def _tc_final(p, h2, dinv, b2, g2, be2, batch2d, Wl, bl):
    def body(p_ref, h_ref, dinv_ref, b_ref, g_ref, be_ref, bat_ref,
             wl_ref, bl_ref, o_ref):
        dv = dinv_ref[...]
        a = _combine(p_ref, h_ref[...], dv, b_ref[...])
        y = _bn_relu(a, g_ref[...], be_ref[...])
        oh = (bat_ref[...] == lax.broadcasted_iota(jnp.int32, (N, G), 1)
              ).astype(jnp.float32)
        dnums = (((0,), (0,)), ((), ()))
        sums = lax.dot_general(oh, y, dnums,
                               preferred_element_type=jnp.float32)
        cnt = lax.dot_general(oh, jnp.ones((N, 1), jnp.float32), dnums,
                              preferred_element_type=jnp.float32)
        pooled = sums / jnp.maximum(cnt, 1.0)
        o_ref[...] = jnp.dot(pooled, wl_ref[...],
                             preferred_element_type=jnp.float32) + bl_ref[...]

    return pl.pallas_call(
        body,
        out_shape=jax.ShapeDtypeStruct((G, C), jnp.float32),
    )(p, h2, dinv, b2, g2, be2, batch2d, Wl, bl)


@jax.jit
def kernel(x, edge_index, batch, W1, b1, g1, be1, W2, b2, g2, be2, Wl, bl):
    src2d = _pad_edges(edge_index[0], None)
    dst2d = _pad_edges(edge_index[1], N)

    degp = _sc_degree(dst2d)
    degT = degp.T
    h1, u1A, u1B, dinv = _tc_first(x, W1, degT)
    p1 = _sc_propagate(u1A, u1B, src2d, dst2d)
    h2, u2A, u2B = _tc_mid(p1, h1, dinv,
                           b1.reshape(1, D), g1.reshape(1, D),
                           be1.reshape(1, D), W2)
    p2 = _sc_propagate(u2A, u2B, src2d, dst2d)
    return _tc_final(p2, h2, dinv,
                     b2.reshape(1, D), g2.reshape(1, D), be2.reshape(1, D),
                     batch.reshape(N, 1), Wl, bl.reshape(1, C))


# speedup vs baseline: 22.2427x; 1.0791x over previous
import functools

import jax
import jax.numpy as jnp
from jax import lax
from jax.experimental import pallas as pl
from jax.experimental.pallas import tpu as pltpu
from jax.experimental.pallas import tpu_sc as plsc

N = 10000
E = 320000
D = 128
HD = D // 2
G = 64
C = 40
EPS = 1e-5

NC = 2
NS = 16
NW = NC * NS
L = 16

CHUNK = 128
CPT = 80
EPAD = NW * CPT * CHUNK
ACCN = 10112
PADROWS = ACCN - N
RPT = ACCN // NS
OUT_TAIL = N - RPT * (NS - 1)
ZR = 8
NBUF = 8


def _mesh():
    return plsc.VectorSubcoreMesh(
        core_axis_name="c", subcore_axis_name="s", num_cores=NC, num_subcores=NS)


def _pad_edges(idx, sink):
    pad = (sink + jnp.arange(EPAD - E, dtype=idx.dtype) % PADROWS
           if sink is not None else
           (jnp.arange(EPAD - E, dtype=idx.dtype) * 131) % N)
    return jnp.concatenate([idx, pad]).reshape(EPAD // CHUNK, CHUNK)


def _sc_degree(dst2d):
    @functools.partial(
        pl.kernel,
        out_type=jax.ShapeDtypeStruct((NC, N), jnp.float32),
        mesh=_mesh(),
        compiler_params=pltpu.CompilerParams(use_tc_tiling_on_sc=False),
        scratch_types=[
            pltpu.VMEM((CPT, CHUNK), jnp.int32),
            pltpu.VMEM((CHUNK,), jnp.float32),
            pltpu.VMEM((RPT,), jnp.float32),
            pltpu.VMEM_SHARED((ACCN,), jnp.float32),
        ],
    )
    def k(dst_hbm, out_hbm, dsts, ones, zbuf, acc):
        c = lax.axis_index("c")
        s = lax.axis_index("s")
        wid = c * NS + s

        one16 = jnp.ones((L,), jnp.float32)
        zero16 = jnp.zeros((L,), jnp.float32)
        for kk in range(CHUNK // L):
            ones[pl.ds(kk * L, L)] = one16
        for kk in range(RPT // L):
            zbuf[pl.ds(kk * L, L)] = zero16

        pltpu.sync_copy(zbuf, acc.at[pl.ds(s * RPT, RPT)])
        plsc.subcore_barrier()

        pltpu.sync_copy(dst_hbm.at[pl.ds(wid * CPT, CPT)], dsts)

        def body(j, carry):
            pltpu.sync_copy(ones, acc.at[dsts.at[j]], add=True)
            return carry

        lax.fori_loop(0, CPT, body, 0)
        plsc.subcore_barrier()

        @pl.when(s < NS - 1)
        def _():
            pltpu.sync_copy(acc.at[pl.ds(s * RPT, RPT)],
                            out_hbm.at[c, pl.ds(s * RPT, RPT)])

        @pl.when(s == NS - 1)
        def _():
            pltpu.sync_copy(acc.at[pl.ds((NS - 1) * RPT, OUT_TAIL)],
                            out_hbm.at[c, pl.ds((NS - 1) * RPT, OUT_TAIL)])

    return k(dst2d)


def _sc_propagate(uA, uB, src2d, dst2d):
    @functools.partial(
        pl.kernel,
        out_type=jax.ShapeDtypeStruct((2, NC, N, HD), jnp.float32),
        mesh=_mesh(),
        compiler_params=pltpu.CompilerParams(use_tc_tiling_on_sc=False),
        scratch_types=[
            pltpu.VMEM((CPT, CHUNK), jnp.int32),
            pltpu.VMEM((CPT, CHUNK), jnp.int32),
            pltpu.VMEM((NBUF, CHUNK, HD), jnp.float32),
            pltpu.VMEM((ZR, HD), jnp.float32),
            pltpu.VMEM_SHARED((ACCN, HD), jnp.float32),
            pltpu.SemaphoreType.DMA,
        ] + [pltpu.SemaphoreType.DMA] * NBUF,
    )
    def k(uA_hbm, uB_hbm, src_hbm, dst_hbm, out_hbm, srcs, dsts, rows, zbuf,
          acc, ssem, *gsems):
        c = lax.axis_index("c")
        s = lax.axis_index("s")
        wid = c * NS + s

        zero16 = jnp.zeros((L,), jnp.float32)

        def zrow(r, carry):
            for kk in range(HD // L):
                zbuf[r, pl.ds(kk * L, L)] = zero16
            return carry

        lax.fori_loop(0, ZR, zrow, 0)

        pltpu.sync_copy(src_hbm.at[pl.ds(wid * CPT, CPT)], srcs)
        pltpu.sync_copy(dst_hbm.at[pl.ds(wid * CPT, CPT)], dsts)

        for half, u_hbm in ((0, uA_hbm), (1, uB_hbm)):
            def zcopy(t, carry):
                pltpu.sync_copy(zbuf, acc.at[pl.ds(s * RPT + t * ZR, ZR)])
                return carry

            lax.fori_loop(0, RPT // ZR, zcopy, 0)
            plsc.subcore_barrier()

            def body(it, carry):
                j0 = it * NBUF
                for b in range(NBUF):
                    pltpu.async_copy(
                        u_hbm.at[srcs.at[j0 + b]], rows.at[b], gsems[b])
                for b in range(NBUF):
                    pltpu.make_async_copy(
                        u_hbm.at[srcs.at[j0 + b]], rows.at[b], gsems[b]).wait()
                for b in range(NBUF):
                    pltpu.async_copy(
                        rows.at[b], acc.at[dsts.at[j0 + b]], ssem, add=True)
                for b in range(NBUF):
                    pltpu.make_async_copy(
                        rows.at[b], acc.at[dsts.at[j0 + b]], ssem).wait()
                return carry

            lax.fori_loop(0, CPT // NBUF, body, 0)
            plsc.subcore_barrier()

            @pl.when(s < NS - 1)
            def _():
                pltpu.sync_copy(acc.at[pl.ds(s * RPT, RPT)],
                                out_hbm.at[half, c, pl.ds(s * RPT, RPT)])

            @pl.when(s == NS - 1)
            def _():
                pltpu.sync_copy(
                    acc.at[pl.ds((NS - 1) * RPT, OUT_TAIL)],
                    out_hbm.at[half, c, pl.ds((NS - 1) * RPT, OUT_TAIL)])

            plsc.subcore_barrier()

    return k(uA, uB, src2d, dst2d)


def _split(u):
    return u[:, :HD], u[:, HD:]


def _combine(p_ref, h, dv, b):
    sA = p_ref[0, 0] + p_ref[0, 1]
    sB = p_ref[1, 0] + p_ref[1, 1]
    return dv * jnp.concatenate([sA, sB], axis=1) + (dv * dv) * h + b


def _tc_first(x, W1, degT):
    def body(x_ref, w_ref, deg_ref, h_ref, uA_ref, uB_ref, dinv_ref):
        deg = jnp.sum(deg_ref[...], axis=1, keepdims=True) + 1.0
        dinv = lax.rsqrt(deg)
        h = jnp.dot(x_ref[...], w_ref[...], preferred_element_type=jnp.float32)
        u = h * dinv
        h_ref[...] = h
        uA_ref[...], uB_ref[...] = _split(u)
        dinv_ref[...] = dinv

    return pl.pallas_call(
        body,
        out_shape=(
            jax.ShapeDtypeStruct((N, D), jnp.float32),
            jax.ShapeDtypeStruct((N, HD), jnp.float32),
            jax.ShapeDtypeStruct((N, HD), jnp.float32),
            jax.ShapeDtypeStruct((N, 1), jnp.float32),
        ),
    )(x, W1, degT)


def _bn_relu(a, g, be):
    mu = jnp.mean(a, axis=0, keepdims=True)
    d = a - mu
    var = jnp.mean(d * d, axis=0, keepdims=True)
    return jnp.maximum(d * lax.rsqrt(var + EPS) * g + be, 0.0)


def _tc_mid(p, h1, dinv, b1, g1, be1, W2):
    def body(p_ref, h_ref, dinv_ref, b_ref, g_ref, be_ref, w_ref,
             h2_ref, u2A_ref, u2B_ref):
        dv = dinv_ref[...]
        a = _combine(p_ref, h_ref[...], dv, b_ref[...])
        y = _bn_relu(a, g_ref[...], be_ref[...])
        h2 = jnp.dot(y, w_ref[...], preferred_element_type=jnp.float32)
        u2 = h2 * dv
        h2_ref[...] = h2
        u2A_ref[...], u2B_ref[...] = _split(u2)

    return pl.pallas_call(
        body,
        out_shape=(
            jax.ShapeDtypeStruct((N, D), jnp.float32),
            jax.ShapeDtypeStruct((N, HD), jnp.float32),
            jax.ShapeDtypeStruct((N, HD), jnp.float32),
        ),
    )(p, h1, dinv, b1, g1, be1, W2)


def _tc_final(p, h2, dinv, b2, g2, be2, batch2d, Wl, bl):
    def body(p_ref, h_ref, dinv_ref, b_ref, g_ref, be_ref, bat_ref,
             wl_ref, bl_ref, o_ref):
        dv = dinv_ref[...]
        a = _combine(p_ref, h_ref[...], dv, b_ref[...])
        y = _bn_relu(a, g_ref[...], be_ref[...])
        oh = (bat_ref[...] == lax.broadcasted_iota(jnp.int32, (N, G), 1)
              ).astype(jnp.float32)
        dnums = (((0,), (0,)), ((), ()))
        sums = lax.dot_general(oh, y, dnums,
                               preferred_element_type=jnp.float32)
        cnt = lax.dot_general(oh, jnp.ones((N, 1), jnp.float32), dnums,
                              preferred_element_type=jnp.float32)
        pooled = sums / jnp.maximum(cnt, 1.0)
        o_ref[...] = jnp.dot(pooled, wl_ref[...],
                             preferred_element_type=jnp.float32) + bl_ref[...]

    return pl.pallas_call(
        body,
        out_shape=jax.ShapeDtypeStruct((G, C), jnp.float32),
    )(p, h2, dinv, b2, g2, be2, batch2d, Wl, bl)


@jax.jit
def kernel(x, edge_index, batch, W1, b1, g1, be1, W2, b2, g2, be2, Wl, bl):
    src2d = _pad_edges(edge_index[0], None)
    dst2d = _pad_edges(edge_index[1], N)

    degp = _sc_degree(dst2d)
    degT = degp.T
    h1, u1A, u1B, dinv = _tc_first(x, W1, degT)
    p1 = _sc_propagate(u1A, u1B, src2d, dst2d)
    h2, u2A, u2B = _tc_mid(p1, h1, dinv,
                           b1.reshape(1, D), g1.reshape(1, D),
                           be1.reshape(1, D), W2)
    p2 = _sc_propagate(u2A, u2B, src2d, dst2d)
    return _tc_final(p2, h2, dinv,
                     b2.reshape(1, D), g2.reshape(1, D), be2.reshape(1, D),
                     batch.reshape(N, 1), Wl, bl.reshape(1, C))


# speedup vs baseline: 25.4319x; 1.1434x over previous
import functools

import jax
import jax.numpy as jnp
from jax import lax
from jax.experimental import pallas as pl
from jax.experimental.pallas import tpu as pltpu
from jax.experimental.pallas import tpu_sc as plsc

N = 10000
E = 320000
D = 128
HD = D // 2
G = 64
C = 40
EPS = 1e-5

NC = 2
NS = 16
NW = NC * NS
L = 16

CHUNK = 128
CPT = 80
EPAD = NW * CPT * CHUNK
ACCN = 10112
PADROWS = ACCN - N
RPT = ACCN // NS
OUT_TAIL = N - RPT * (NS - 1)
ZR = 8
NBUF = 2
SPT = 40


def _mesh():
    return plsc.VectorSubcoreMesh(
        core_axis_name="c", subcore_axis_name="s", num_cores=NC, num_subcores=NS)


def _pad_edges(idx, sink):
    pad = (sink + jnp.arange(EPAD - E, dtype=idx.dtype) % PADROWS
           if sink is not None else
           (jnp.arange(EPAD - E, dtype=idx.dtype) * 131) % N)
    return jnp.concatenate([idx, pad]).reshape(EPAD // CHUNK, CHUNK)


def _sc_degree(dst2d):
    @functools.partial(
        pl.kernel,
        out_type=jax.ShapeDtypeStruct((NC, N), jnp.float32),
        mesh=_mesh(),
        compiler_params=pltpu.CompilerParams(use_tc_tiling_on_sc=False),
        scratch_types=[
            pltpu.VMEM((CPT, CHUNK), jnp.int32),
            pltpu.VMEM((CHUNK,), jnp.float32),
            pltpu.VMEM((RPT,), jnp.float32),
            pltpu.VMEM_SHARED((ACCN,), jnp.float32),
        ],
    )
    def k(dst_hbm, out_hbm, dsts, ones, zbuf, acc):
        c = lax.axis_index("c")
        s = lax.axis_index("s")
        wid = c * NS + s

        one16 = jnp.ones((L,), jnp.float32)
        zero16 = jnp.zeros((L,), jnp.float32)
        for kk in range(CHUNK // L):
            ones[pl.ds(kk * L, L)] = one16
        for kk in range(RPT // L):
            zbuf[pl.ds(kk * L, L)] = zero16

        pltpu.sync_copy(zbuf, acc.at[pl.ds(s * RPT, RPT)])
        plsc.subcore_barrier()

        pltpu.sync_copy(dst_hbm.at[pl.ds(wid * CPT, CPT)], dsts)

        def body(j, carry):
            pltpu.sync_copy(ones, acc.at[dsts.at[j]], add=True)
            return carry

        lax.fori_loop(0, CPT, body, 0)
        plsc.subcore_barrier()

        @pl.when(s < NS - 1)
        def _():
            pltpu.sync_copy(acc.at[pl.ds(s * RPT, RPT)],
                            out_hbm.at[c, pl.ds(s * RPT, RPT)])

        @pl.when(s == NS - 1)
        def _():
            pltpu.sync_copy(acc.at[pl.ds((NS - 1) * RPT, OUT_TAIL)],
                            out_hbm.at[c, pl.ds((NS - 1) * RPT, OUT_TAIL)])

    return k(dst2d)


def _sc_propagate(u, src2d, dst2d):
    @functools.partial(
        pl.kernel,
        out_type=jax.ShapeDtypeStruct((NC, N, D), jnp.float32),
        mesh=_mesh(),
        compiler_params=pltpu.CompilerParams(use_tc_tiling_on_sc=False),
        scratch_types=[
            pltpu.VMEM((SPT, CHUNK), jnp.int32),
            pltpu.VMEM((SPT, CHUNK), jnp.int32),
            pltpu.VMEM((NBUF, CHUNK, D), jnp.float32),
            pltpu.VMEM((ZR, D), jnp.float32),
            pltpu.VMEM_SHARED((ACCN, D), jnp.float32),
        ] + [pltpu.SemaphoreType.DMA] * (2 * NBUF),
    )
    def k(u_hbm, src_hbm, dst_hbm, out_hbm, srcs, dsts, rows, zbuf,
          acc, *sems):
        gsems = sems[:NBUF]
        ssems = sems[NBUF:]
        c = lax.axis_index("c")
        s = lax.axis_index("s")
        wid = c * NS + s

        zero16 = jnp.zeros((L,), jnp.float32)

        def zrow(r, carry):
            for kk in range(D // L):
                zbuf[r, pl.ds(kk * L, L)] = zero16
            return carry

        lax.fori_loop(0, ZR, zrow, 0)

        def zcopy(t, carry):
            pltpu.sync_copy(zbuf, acc.at[pl.ds(s * RPT + t * ZR, ZR)])
            return carry

        lax.fori_loop(0, RPT // ZR, zcopy, 0)
        plsc.subcore_barrier()

        for seg in range(CPT // SPT):
            c0 = wid * CPT + seg * SPT
            pltpu.sync_copy(src_hbm.at[pl.ds(c0, SPT)], srcs)
            pltpu.sync_copy(dst_hbm.at[pl.ds(c0, SPT)], dsts)

            def body(it, carry):
                j0 = it * NBUF
                gh = [pltpu.async_copy(
                    u_hbm.at[srcs.at[j0 + b]], rows.at[b], gsems[b])
                    for b in range(NBUF)]
                for h in gh:
                    h.wait()
                sh = [pltpu.async_copy(
                    rows.at[b], acc.at[dsts.at[j0 + b]], ssems[b], add=True)
                    for b in range(NBUF)]
                for h in sh:
                    h.wait()
                return carry

            lax.fori_loop(0, SPT // NBUF, body, 0)

        plsc.subcore_barrier()

        @pl.when(s < NS - 1)
        def _():
            pltpu.sync_copy(acc.at[pl.ds(s * RPT, RPT)],
                            out_hbm.at[c, pl.ds(s * RPT, RPT)])

        @pl.when(s == NS - 1)
        def _():
            pltpu.sync_copy(
                acc.at[pl.ds((NS - 1) * RPT, OUT_TAIL)],
                out_hbm.at[c, pl.ds((NS - 1) * RPT, OUT_TAIL)])

    return k(u, src2d, dst2d)


def _combine(p_ref, h, dv, b):
    return dv * (p_ref[0] + p_ref[1]) + (dv * dv) * h + b


def _tc_first(x, W1, degT):
    def body(x_ref, w_ref, deg_ref, h_ref, u_ref, dinv_ref):
        deg = jnp.sum(deg_ref[...], axis=1, keepdims=True) + 1.0
        dinv = lax.rsqrt(deg)
        h = jnp.dot(x_ref[...], w_ref[...], preferred_element_type=jnp.float32)
        h_ref[...] = h
        u_ref[...] = h * dinv
        dinv_ref[...] = dinv

    return pl.pallas_call(
        body,
        out_shape=(
            jax.ShapeDtypeStruct((N, D), jnp.float32),
            jax.ShapeDtypeStruct((N, D), jnp.float32),
            jax.ShapeDtypeStruct((N, 1), jnp.float32),
        ),
    )(x, W1, degT)


def _bn_relu(a, g, be):
    mu = jnp.mean(a, axis=0, keepdims=True)
    d = a - mu
    var = jnp.mean(d * d, axis=0, keepdims=True)
    return jnp.maximum(d * lax.rsqrt(var + EPS) * g + be, 0.0)


def _tc_mid(p, h1, dinv, b1, g1, be1, W2):
    def body(p_ref, h_ref, dinv_ref, b_ref, g_ref, be_ref, w_ref,
             h2_ref, u2_ref):
        dv = dinv_ref[...]
        a = _combine(p_ref, h_ref[...], dv, b_ref[...])
        y = _bn_relu(a, g_ref[...], be_ref[...])
        h2 = jnp.dot(y, w_ref[...], preferred_element_type=jnp.float32)
        h2_ref[...] = h2
        u2_ref[...] = h2 * dv

    return pl.pallas_call(
        body,
        out_shape=(
            jax.ShapeDtypeStruct((N, D), jnp.float32),
            jax.ShapeDtypeStruct((N, D), jnp.float32),
        ),
    )(p, h1, dinv, b1, g1, be1, W2)


def _tc_final(p, h2, dinv, b2, g2, be2, batch2d, Wl, bl):
    def body(p_ref, h_ref, dinv_ref, b_ref, g_ref, be_ref, bat_ref,
             wl_ref, bl_ref, o_ref):
        dv = dinv_ref[...]
        a = _combine(p_ref, h_ref[...], dv, b_ref[...])
        y = _bn_relu(a, g_ref[...], be_ref[...])
        oh = (bat_ref[...] == lax.broadcasted_iota(jnp.int32, (N, G), 1)
              ).astype(jnp.float32)
        dnums = (((0,), (0,)), ((), ()))
        sums = lax.dot_general(oh, y, dnums,
                               preferred_element_type=jnp.float32)
        cnt = lax.dot_general(oh, jnp.ones((N, 1), jnp.float32), dnums,
                              preferred_element_type=jnp.float32)
        pooled = sums / jnp.maximum(cnt, 1.0)
        o_ref[...] = jnp.dot(pooled, wl_ref[...],
                             preferred_element_type=jnp.float32) + bl_ref[...]

    return pl.pallas_call(
        body,
        out_shape=jax.ShapeDtypeStruct((G, C), jnp.float32),
    )(p, h2, dinv, b2, g2, be2, batch2d, Wl, bl)


@jax.jit
def kernel(x, edge_index, batch, W1, b1, g1, be1, W2, b2, g2, be2, Wl, bl):
    src2d = _pad_edges(edge_index[0], None)
    dst2d = _pad_edges(edge_index[1], N)

    degp = _sc_degree(dst2d)
    degT = degp.T
    h1, u1, dinv = _tc_first(x, W1, degT)
    p1 = _sc_propagate(u1, src2d, dst2d)
    h2, u2 = _tc_mid(p1, h1, dinv,
                     b1.reshape(1, D), g1.reshape(1, D),
                     be1.reshape(1, D), W2)
    p2 = _sc_propagate(u2, src2d, dst2d)
    return _tc_final(p2, h2, dinv,
                     b2.reshape(1, D), g2.reshape(1, D), be2.reshape(1, D),
                     batch.reshape(N, 1), Wl, bl.reshape(1, C))


# speedup vs baseline: 25.6759x; 1.0096x over previous
import functools

import jax
import jax.numpy as jnp
from jax import lax
from jax.experimental import pallas as pl
from jax.experimental.pallas import tpu as pltpu
from jax.experimental.pallas import tpu_sc as plsc

N = 10000
E = 320000
D = 128
HD = D // 2
G = 64
C = 40
EPS = 1e-5

NC = 2
NS = 16
NW = NC * NS
L = 16

CHUNK = 128
CPT = 80
EPAD = NW * CPT * CHUNK
ACCN = 10112
PADROWS = ACCN - N
RPT = ACCN // NS
OUT_TAIL = N - RPT * (NS - 1)
ZR = 8
NBUF = 2
SPT = 40


def _mesh():
    return plsc.VectorSubcoreMesh(
        core_axis_name="c", subcore_axis_name="s", num_cores=NC, num_subcores=NS)


def _pad_edges(idx, sink):
    pad = (sink + jnp.arange(EPAD - E, dtype=idx.dtype) % PADROWS
           if sink is not None else
           (jnp.arange(EPAD - E, dtype=idx.dtype) * 131) % N)
    return jnp.concatenate([idx, pad]).reshape(EPAD // CHUNK, CHUNK)


def _sc_degree(dst2d):
    @functools.partial(
        pl.kernel,
        out_type=jax.ShapeDtypeStruct((NC, N), jnp.float32),
        mesh=_mesh(),
        compiler_params=pltpu.CompilerParams(use_tc_tiling_on_sc=False),
        scratch_types=[
            pltpu.VMEM((CPT, CHUNK), jnp.int32),
            pltpu.VMEM((CHUNK,), jnp.float32),
            pltpu.VMEM((RPT,), jnp.float32),
            pltpu.VMEM_SHARED((ACCN,), jnp.float32),
        ],
    )
    def k(dst_hbm, out_hbm, dsts, ones, zbuf, acc):
        c = lax.axis_index("c")
        s = lax.axis_index("s")
        wid = c * NS + s

        one16 = jnp.ones((L,), jnp.float32)
        zero16 = jnp.zeros((L,), jnp.float32)
        for kk in range(CHUNK // L):
            ones[pl.ds(kk * L, L)] = one16
        for kk in range(RPT // L):
            zbuf[pl.ds(kk * L, L)] = zero16

        pltpu.sync_copy(zbuf, acc.at[pl.ds(s * RPT, RPT)])
        plsc.subcore_barrier()

        pltpu.sync_copy(dst_hbm.at[pl.ds(wid * CPT, CPT)], dsts)

        def body(j, carry):
            pltpu.sync_copy(ones, acc.at[dsts.at[j]], add=True)
            return carry

        lax.fori_loop(0, CPT, body, 0)
        plsc.subcore_barrier()

        @pl.when(s < NS - 1)
        def _():
            pltpu.sync_copy(acc.at[pl.ds(s * RPT, RPT)],
                            out_hbm.at[c, pl.ds(s * RPT, RPT)])

        @pl.when(s == NS - 1)
        def _():
            pltpu.sync_copy(acc.at[pl.ds((NS - 1) * RPT, OUT_TAIL)],
                            out_hbm.at[c, pl.ds((NS - 1) * RPT, OUT_TAIL)])

    return k(dst2d)


def _sc_propagate(u, src2d, dst2d):
    @functools.partial(
        pl.kernel,
        out_type=jax.ShapeDtypeStruct((NC, N, D), jnp.float32),
        mesh=_mesh(),
        compiler_params=pltpu.CompilerParams(use_tc_tiling_on_sc=False),
        scratch_types=[
            pltpu.VMEM((SPT, CHUNK), jnp.int32),
            pltpu.VMEM((SPT, CHUNK), jnp.int32),
            pltpu.VMEM((NBUF, CHUNK, D), jnp.float32),
            pltpu.VMEM((ZR, D), jnp.float32),
            pltpu.VMEM_SHARED((ACCN, D), jnp.float32),
        ] + [pltpu.SemaphoreType.DMA] * (2 * NBUF),
    )
    def k(u_hbm, src_hbm, dst_hbm, out_hbm, srcs, dsts, rows, zbuf,
          acc, *sems):
        gsems = sems[:NBUF]
        ssems = sems[NBUF:]
        c = lax.axis_index("c")
        s = lax.axis_index("s")
        wid = c * NS + s

        zero16 = jnp.zeros((L,), jnp.float32)

        def zrow(r, carry):
            for kk in range(D // L):
                zbuf[r, pl.ds(kk * L, L)] = zero16
            return carry

        lax.fori_loop(0, ZR, zrow, 0)

        def zcopy(t, carry):
            pltpu.sync_copy(zbuf, acc.at[pl.ds(s * RPT + t * ZR, ZR)])
            return carry

        lax.fori_loop(0, RPT // ZR, zcopy, 0)
        plsc.subcore_barrier()

        for seg in range(CPT // SPT):
            c0 = wid * CPT + seg * SPT
            pltpu.sync_copy(src_hbm.at[pl.ds(c0, SPT)], srcs)
            pltpu.sync_copy(dst_hbm.at[pl.ds(c0, SPT)], dsts)

            def body(it, carry):
                j0 = it * NBUF
                gh = [pltpu.async_copy(
                    u_hbm.at[srcs.at[j0 + b]], rows.at[b], gsems[b])
                    for b in range(NBUF)]
                sh = []
                for b in range(NBUF):
                    gh[b].wait()
                    sh.append(pltpu.async_copy(
                        rows.at[b], acc.at[dsts.at[j0 + b]], ssems[b],
                        add=True))
                for h in sh:
                    h.wait()
                return carry

            lax.fori_loop(0, SPT // NBUF, body, 0)

        plsc.subcore_barrier()

        @pl.when(s < NS - 1)
        def _():
            pltpu.sync_copy(acc.at[pl.ds(s * RPT, RPT)],
                            out_hbm.at[c, pl.ds(s * RPT, RPT)])

        @pl.when(s == NS - 1)
        def _():
            pltpu.sync_copy(
                acc.at[pl.ds((NS - 1) * RPT, OUT_TAIL)],
                out_hbm.at[c, pl.ds((NS - 1) * RPT, OUT_TAIL)])

    return k(u, src2d, dst2d)


def _combine(p_ref, h, dv, b):
    return dv * (p_ref[0] + p_ref[1]) + (dv * dv) * h + b


def _tc_first(x, W1, degT):
    def body(x_ref, w_ref, deg_ref, h_ref, u_ref, dinv_ref):
        deg = jnp.sum(deg_ref[...], axis=1, keepdims=True) + 1.0
        dinv = lax.rsqrt(deg)
        h = jnp.dot(x_ref[...], w_ref[...], preferred_element_type=jnp.float32)
        h_ref[...] = h
        u_ref[...] = h * dinv
        dinv_ref[...] = dinv

    return pl.pallas_call(
        body,
        out_shape=(
            jax.ShapeDtypeStruct((N, D), jnp.float32),
            jax.ShapeDtypeStruct((N, D), jnp.float32),
            jax.ShapeDtypeStruct((N, 1), jnp.float32),
        ),
    )(x, W1, degT)


def _bn_relu(a, g, be):
    mu = jnp.mean(a, axis=0, keepdims=True)
    d = a - mu
    var = jnp.mean(d * d, axis=0, keepdims=True)
    return jnp.maximum(d * lax.rsqrt(var + EPS) * g + be, 0.0)


def _tc_mid(p, h1, dinv, b1, g1, be1, W2):
    def body(p_ref, h_ref, dinv_ref, b_ref, g_ref, be_ref, w_ref,
             h2_ref, u2_ref):
        dv = dinv_ref[...]
        a = _combine(p_ref, h_ref[...], dv, b_ref[...])
        y = _bn_relu(a, g_ref[...], be_ref[...])
        h2 = jnp.dot(y, w_ref[...], preferred_element_type=jnp.float32)
        h2_ref[...] = h2
        u2_ref[...] = h2 * dv

    return pl.pallas_call(
        body,
        out_shape=(
            jax.ShapeDtypeStruct((N, D), jnp.float32),
            jax.ShapeDtypeStruct((N, D), jnp.float32),
        ),
    )(p, h1, dinv, b1, g1, be1, W2)


def _tc_final(p, h2, dinv, b2, g2, be2, batch2d, Wl, bl):
    def body(p_ref, h_ref, dinv_ref, b_ref, g_ref, be_ref, bat_ref,
             wl_ref, bl_ref, o_ref):
        dv = dinv_ref[...]
        a = _combine(p_ref, h_ref[...], dv, b_ref[...])
        y = _bn_relu(a, g_ref[...], be_ref[...])
        oh = (bat_ref[...] == lax.broadcasted_iota(jnp.int32, (N, G), 1)
              ).astype(jnp.float32)
        dnums = (((0,), (0,)), ((), ()))
        sums = lax.dot_general(oh, y, dnums,
                               preferred_element_type=jnp.float32)
        cnt = lax.dot_general(oh, jnp.ones((N, 1), jnp.float32), dnums,
                              preferred_element_type=jnp.float32)
        pooled = sums / jnp.maximum(cnt, 1.0)
        o_ref[...] = jnp.dot(pooled, wl_ref[...],
                             preferred_element_type=jnp.float32) + bl_ref[...]

    return pl.pallas_call(
        body,
        out_shape=jax.ShapeDtypeStruct((G, C), jnp.float32),
    )(p, h2, dinv, b2, g2, be2, batch2d, Wl, bl)


@jax.jit
def kernel(x, edge_index, batch, W1, b1, g1, be1, W2, b2, g2, be2, Wl, bl):
    src2d = _pad_edges(edge_index[0], None)
    dst2d = _pad_edges(edge_index[1], N)

    degp = _sc_degree(dst2d)
    degT = degp.T
    h1, u1, dinv = _tc_first(x, W1, degT)
    p1 = _sc_propagate(u1, src2d, dst2d)
    h2, u2 = _tc_mid(p1, h1, dinv,
                     b1.reshape(1, D), g1.reshape(1, D),
                     be1.reshape(1, D), W2)
    p2 = _sc_propagate(u2, src2d, dst2d)
    return _tc_final(p2, h2, dinv,
                     b2.reshape(1, D), g2.reshape(1, D), be2.reshape(1, D),
                     batch.reshape(N, 1), Wl, bl.reshape(1, C))
